# baseline (device time: 25937 ns/iter reference)
import jax
import jax.numpy as jnp
from jax import lax
from jax.experimental import pallas as pl
from jax.experimental.pallas import tpu as pltpu

YS, ZS = 4, 4
CH = 64


def kernel(Q, K, V):
    b, s, h, d = Q.shape
    bs, hd = b * s, h * d
    rows = 2 * bs
    scale = d ** -0.5

    def body(q_ref, k_ref, v_ref, out_ref, kv_send, kv_rem, qb_ref,
             o_acc, l_acc, x_send_sems, fwd_send_sems, recv_sems):
        my_x = lax.axis_index("x")
        my_y = lax.axis_index("y")
        my_z = lax.axis_index("z")
        py = lax.rem(my_y, 2)
        pz = lax.rem(my_z, 2)
        o_m = py + 2 * pz
        o_d = 3 - o_m
        o_y = 1 - py + 2 * pz
        o_z = py + 2 * (1 - pz)
        xnbr = (1 - my_x, my_y, my_z)
        yp = (my_x, lax.rem(my_y + 1, YS), my_z)
        ym = (my_x, lax.rem(my_y + YS - 1, YS), my_z)
        zp = (my_x, my_y, lax.rem(my_z + 1, ZS))
        zm = (my_x, my_y, lax.rem(my_z + ZS - 1, ZS))

        barrier_sem = pltpu.get_barrier_semaphore()
        for nb in (xnbr, ym, zm):
            pl.semaphore_signal(
                barrier_sem, inc=1, device_id=nb,
                device_id_type=pl.DeviceIdType.MESH,
            )

        def chunk_row(q, c):
            base = q * 128 + (c % 2) * CH
            return base + (bs if c >= 2 else 0)

        for q in (o_m, o_d):
            for c in range(2):
                r = q * 128 + c * CH
                kv_send[pl.ds(r, CH), :] = (
                    k_ref[pl.ds(r, CH), :].astype(jnp.bfloat16))
                kv_send[pl.ds(bs + r, CH), :] = (
                    v_ref[pl.ds(r, CH), :].astype(jnp.bfloat16))

        pl.semaphore_wait(barrier_sem, 3)

        x_rdmas = []
        for qi, q in enumerate((o_m, o_d)):
            for c in range(4):
                i = qi * 4 + c
                sl = pl.ds(chunk_row(q, c), CH)
                rdma = pltpu.make_async_remote_copy(
                    src_ref=kv_send.at[sl],
                    dst_ref=kv_rem.at[sl],
                    send_sem=x_send_sems.at[i],
                    recv_sem=recv_sems.at[i],
                    device_id=xnbr,
                    device_id_type=pl.DeviceIdType.MESH,
                )
                rdma.start()
                x_rdmas.append(rdma)

        for q in (o_y, o_z):
            for c in range(2):
                r = q * 128 + c * CH
                kv_send[pl.ds(r, CH), :] = (
                    k_ref[pl.ds(r, CH), :].astype(jnp.bfloat16))
                kv_send[pl.ds(bs + r, CH), :] = (
                    v_ref[pl.ds(r, CH), :].astype(jnp.bfloat16))
        qb_ref[...] = (q_ref[...] * scale).astype(jnp.bfloat16)

        fwds = []

        def issue_fwd(c):
            x_rdmas[c].wait()
            sl = pl.ds(chunk_row(o_m, c), CH)
            for t, (nb, sem_i, base) in enumerate(
                    ((yp, c, 8), (zp, 4 + c, 12))):
                fwd = pltpu.make_async_remote_copy(
                    src_ref=kv_rem.at[sl],
                    dst_ref=kv_rem.at[sl],
                    send_sem=fwd_send_sems.at[sem_i],
                    recv_sem=recv_sems.at[base + c],
                    device_id=nb,
                    device_id_type=pl.DeviceIdType.MESH,
                )
                fwd.start()
                fwds.append(fwd)

        for i in range(b * h):
            bi, hi = i // h, i % h
            r0, c0 = bi * s, hi * d
            qb = qb_ref[r0:r0 + s, c0:c0 + d]
            kb = kv_send[r0:r0 + s, c0:c0 + d]
            vb = kv_send[bs + r0:bs + r0 + s, c0:c0 + d]
            s0 = lax.dot_general(
                qb, kb, (((1,), (1,)), ((), ())),
                preferred_element_type=jnp.float32,
            )
            p0 = jnp.exp(s0)
            l_acc[r0:r0 + s, hi:hi + 1] = jnp.sum(p0, axis=1,
                                                  keepdims=True)
            o_acc[r0:r0 + s, c0:c0 + d] = jnp.dot(
                p0.astype(jnp.bfloat16), vb,
                preferred_element_type=jnp.float32,
            )
            if i in (6, 8, 10, 12):
                issue_fwd((i - 6) // 2)

        def process_quarter(q):
            rk = q * 128
            rq = (q // 2) * s
            for hi in range(h):
                c0 = hi * d
                qb = qb_ref[pl.ds(rq, s), c0:c0 + d]
                kb = kv_rem[pl.ds(rk, 128), c0:c0 + d]
                vb = kv_rem[pl.ds(bs + rk, 128), c0:c0 + d]
                s1 = lax.dot_general(
                    qb, kb, (((1,), (1,)), ((), ())),
                    preferred_element_type=jnp.float32,
                )
                p1 = jnp.exp(s1)
                l_acc[pl.ds(rq, s), hi:hi + 1] += jnp.sum(
                    p1, axis=1, keepdims=True)
                o_acc[pl.ds(rq, s), c0:c0 + d] += jnp.dot(
                    p1.astype(jnp.bfloat16), vb,
                    preferred_element_type=jnp.float32,
                )

        process_quarter(o_m)

        for c in range(4):
            pltpu.make_async_remote_copy(
                src_ref=kv_rem.at[pl.ds(chunk_row(o_y, c), CH)],
                dst_ref=kv_rem.at[pl.ds(chunk_row(o_y, c), CH)],
                send_sem=fwd_send_sems.at[c],
                recv_sem=recv_sems.at[8 + c],
                device_id=ym,
                device_id_type=pl.DeviceIdType.MESH,
            ).wait_recv()
        process_quarter(o_y)

        for c in range(4):
            pltpu.make_async_remote_copy(
                src_ref=kv_rem.at[pl.ds(chunk_row(o_z, c), CH)],
                dst_ref=kv_rem.at[pl.ds(chunk_row(o_z, c), CH)],
                send_sem=fwd_send_sems.at[4 + c],
                recv_sem=recv_sems.at[12 + c],
                device_id=zm,
                device_id_type=pl.DeviceIdType.MESH,
            ).wait_recv()
        process_quarter(o_z)

        for c in range(4):
            x_rdmas[4 + c].wait()
        process_quarter(o_d)

        for i in range(b * h):
            bi, hi = i // h, i % h
            r0, c0 = bi * s, hi * d
            out_ref[r0:r0 + s, c0:c0 + d] = (
                o_acc[r0:r0 + s, c0:c0 + d]
                / l_acc[r0:r0 + s, hi:hi + 1])

        for f in fwds:
            f.wait_send()

    out2 = pl.pallas_call(
        body,
        out_shape=jax.ShapeDtypeStruct((bs, hd), jnp.float32),
        in_specs=[
            pl.BlockSpec(memory_space=pltpu.VMEM),
            pl.BlockSpec(memory_space=pltpu.VMEM),
            pl.BlockSpec(memory_space=pltpu.VMEM),
        ],
        out_specs=pl.BlockSpec(memory_space=pltpu.VMEM),
        scratch_shapes=[
            pltpu.VMEM((rows, hd), jnp.bfloat16),
            pltpu.VMEM((rows, hd), jnp.bfloat16),
            pltpu.VMEM((bs, hd), jnp.bfloat16),
            pltpu.VMEM((bs, hd), jnp.float32),
            pltpu.VMEM((bs, h), jnp.float32),
            pltpu.SemaphoreType.DMA((8,)),
            pltpu.SemaphoreType.DMA((8,)),
            pltpu.SemaphoreType.DMA((16,)),
        ],
        compiler_params=pltpu.CompilerParams(collective_id=0),
    )(Q.reshape(bs, hd), K.reshape(bs, hd), V.reshape(bs, hd))
    return out2.reshape(b, s, h, d)
